# baseline (speedup 1.0000x reference)
import functools

import jax
import jax.numpy as jnp
from jax import lax
from jax.experimental import pallas as pl
from jax.experimental.pallas import tpu as pltpu
from jax.experimental.pallas import tpu_sc as plsc

_LANES = 16
_NBUF = 4
_OBUF = 2


def _sc_info():
    try:
        info = plsc.get_sparse_core_info()
        return info.num_cores, info.num_subcores
    except Exception:
        return 2, 16


@functools.cache
def _build(R, V, S, D):
    NC, NS = _sc_info()
    NW = NC * NS
    rows_per_w = R // NW
    assert rows_per_w % S == 0
    nchunks = rows_per_w // S
    assert nchunks % _NBUF == 0
    scale = float(D) ** 0.5

    mesh = plsc.VectorSubcoreMesh(core_axis_name="c", subcore_axis_name="s")

    def body(idx_hbm, tok_hbm, pos_hbm, out_hbm, idx_all,
             in0, in1, in2, in3, o0, o1, pos_all,
             g0, g1, g2, g3, s0, s1):
        cid = lax.axis_index("c")
        sid = lax.axis_index("s")
        wid = sid * NC + cid
        base = wid * rows_per_w

        rows_in = (in0, in1, in2, in3)
        rows_out = (o0, o1)
        gsem = (g0, g1, g2, g3)
        ssem = (s0, s1)

        pltpu.sync_copy(idx_hbm.at[wid], idx_all)
        pltpu.sync_copy(pos_hbm, pos_all)

        def start_gather(c, b):
            pltpu.async_copy(
                tok_hbm.at[idx_all.at[pl.ds(c * S, S)]], rows_in[b], gsem[b])

        for b in range(_NBUF):
            start_gather(b, b)

        def group(c4, carry):
            for b in range(_NBUF):
                o = b % _OBUF
                c = c4 * _NBUF + b
                row0 = base + c * S
                pltpu.make_async_copy(
                    tok_hbm.at[idx_all.at[pl.ds(c * S, S)]], rows_in[b],
                    gsem[b]).wait()

                @pl.when(c >= _OBUF)
                def _():
                    prow0 = row0 - _OBUF * S
                    pltpu.make_async_copy(
                        rows_out[o], out_hbm.at[pl.ds(prow0, S)],
                        ssem[o]).wait()

                src = rows_in[b]
                dst = rows_out[o]

                def row_body(r2, carry2):
                    r = r2 * 2
                    for rr in (r, r + 1):
                        for j in range(D // _LANES):
                            sl = pl.ds(j * _LANES, _LANES)
                            dst[rr, sl] = src[rr, sl] * scale + pos_all[rr, sl]
                    return carry2

                lax.fori_loop(0, S // 2, row_body, 0)

                pltpu.async_copy(dst, out_hbm.at[pl.ds(row0, S)], ssem[o])

                nxt = c + _NBUF

                @pl.when(nxt < nchunks)
                def _():
                    start_gather(nxt, b)
            return carry

        lax.fori_loop(0, nchunks // _NBUF, group, 0)

        for j in range(_OBUF):
            c = nchunks - _OBUF + j
            row0 = base + c * S
            pltpu.make_async_copy(
                rows_out[c % _OBUF], out_hbm.at[pl.ds(row0, S)],
                ssem[c % _OBUF]).wait()

    return pl.kernel(
        body,
        out_type=jax.ShapeDtypeStruct((R, D), jnp.float32),
        mesh=mesh,
        compiler_params=pltpu.CompilerParams(use_tc_tiling_on_sc=False),
        scratch_types=[
            pltpu.VMEM((rows_per_w,), jnp.int32),
            pltpu.VMEM((S, D), jnp.float32),
            pltpu.VMEM((S, D), jnp.float32),
            pltpu.VMEM((S, D), jnp.float32),
            pltpu.VMEM((S, D), jnp.float32),
            pltpu.VMEM((S, D), jnp.float32),
            pltpu.VMEM((S, D), jnp.float32),
            pltpu.VMEM((S, D), jnp.float32),
            pltpu.SemaphoreType.DMA,
            pltpu.SemaphoreType.DMA,
            pltpu.SemaphoreType.DMA,
            pltpu.SemaphoreType.DMA,
            pltpu.SemaphoreType.DMA,
            pltpu.SemaphoreType.DMA,
        ],
    )


def kernel(inputs, token_table, pos_table):
    B, S = inputs.shape
    V, D = token_table.shape
    R = B * S
    NC, NS = _sc_info()
    NW = NC * NS
    idx = inputs.reshape(NW, R // NW).astype(jnp.int32)
    out = _build(R, V, S, D)(idx, token_table, pos_table)
    return out.reshape(B, S, D)


# speedup vs baseline: 1.3299x; 1.3299x over previous
import functools

import jax
import jax.numpy as jnp
from jax import lax
from jax.experimental import pallas as pl
from jax.experimental.pallas import tpu as pltpu
from jax.experimental.pallas import tpu_sc as plsc

_LANES = 16
_NBUF = 4
_OBUF = 2


def _sc_info():
    try:
        info = plsc.get_sparse_core_info()
        return info.num_cores, info.num_subcores
    except Exception:
        return 2, 16


@functools.cache
def _build(R, V, S, D):
    NC, NS = _sc_info()
    NW = NC * NS
    rows_per_w = R // NW
    assert rows_per_w % S == 0
    nchunks = rows_per_w // S
    assert nchunks % _NBUF == 0
    scale = float(D) ** 0.5

    mesh = plsc.VectorSubcoreMesh(core_axis_name="c", subcore_axis_name="s")

    def body(idx_hbm, tok_hbm, pos_hbm, out_hbm, idx_all,
             in0, in1, in2, in3, o0, o1, pos_all,
             g0, g1, g2, g3, s0, s1):
        cid = lax.axis_index("c")
        sid = lax.axis_index("s")
        wid = sid * NC + cid
        base = wid * rows_per_w

        rows_in = (in0, in1, in2, in3)
        rows_out = (o0, o1)
        gsem = (g0, g1, g2, g3)
        ssem = (s0, s1)

        pltpu.sync_copy(idx_hbm.at[wid], idx_all)
        pltpu.sync_copy(pos_hbm, pos_all)

        def start_gather(c, b):
            pltpu.async_copy(
                tok_hbm.at[idx_all.at[pl.ds(c * S, S)]], rows_in[b], gsem[b])

        for b in range(_NBUF):
            start_gather(b, b)

        def group(c4, carry):
            for b in range(_NBUF):
                o = b % _OBUF
                c = c4 * _NBUF + b
                row0 = base + c * S
                pltpu.make_async_copy(
                    tok_hbm.at[idx_all.at[pl.ds(c * S, S)]], rows_in[b],
                    gsem[b]).wait()

                @pl.when(c >= _OBUF)
                def _():
                    prow0 = row0 - _OBUF * S
                    pltpu.make_async_copy(
                        rows_out[o],
                        out_hbm.at[pl.ds(prow0, S), pl.ds(0, D)],
                        ssem[o]).wait()

                src = rows_in[b]
                dst = rows_out[o]

                def row_body(r2, carry2):
                    r = r2 * 2
                    for rr in (r, r + 1):
                        for j in range(D // _LANES):
                            sl = pl.ds(j * _LANES, _LANES)
                            dst[rr, sl] = src[rr, sl] * scale + pos_all[rr, sl]
                    return carry2

                lax.fori_loop(0, S // 2, row_body, 0)

                pltpu.async_copy(
                    dst, out_hbm.at[pl.ds(row0, S), pl.ds(0, D)], ssem[o])

                nxt = c + _NBUF

                @pl.when(nxt < nchunks)
                def _():
                    start_gather(nxt, b)
            return carry

        lax.fori_loop(0, nchunks // _NBUF, group, 0)

        for j in range(_OBUF):
            c = nchunks - _OBUF + j
            row0 = base + c * S
            pltpu.make_async_copy(
                rows_out[c % _OBUF],
                out_hbm.at[pl.ds(row0, S), pl.ds(0, D)],
                ssem[c % _OBUF]).wait()

    return pl.kernel(
        body,
        out_type=jax.ShapeDtypeStruct((R, 2 * D), jnp.float32),
        mesh=mesh,
        compiler_params=pltpu.CompilerParams(use_tc_tiling_on_sc=False),
        scratch_types=[
            pltpu.VMEM((rows_per_w,), jnp.int32),
            pltpu.VMEM((S, D), jnp.float32),
            pltpu.VMEM((S, D), jnp.float32),
            pltpu.VMEM((S, D), jnp.float32),
            pltpu.VMEM((S, D), jnp.float32),
            pltpu.VMEM((S, D), jnp.float32),
            pltpu.VMEM((S, D), jnp.float32),
            pltpu.VMEM((S, D), jnp.float32),
            pltpu.SemaphoreType.DMA,
            pltpu.SemaphoreType.DMA,
            pltpu.SemaphoreType.DMA,
            pltpu.SemaphoreType.DMA,
            pltpu.SemaphoreType.DMA,
            pltpu.SemaphoreType.DMA,
        ],
    )


def kernel(inputs, token_table, pos_table):
    B, S = inputs.shape
    V, D = token_table.shape
    R = B * S
    NC, NS = _sc_info()
    NW = NC * NS
    idx = inputs.reshape(NW, R // NW).astype(jnp.int32)
    out = _build(R, V, S, D)(idx, token_table, pos_table)
    return out[:, :D].reshape(B, S, D)
